# Initial kernel scaffold; baseline (speedup 1.0000x reference)
import jax
import jax.numpy as jnp
from jax.experimental import pallas as pl


def kernel(xyz, new_xyz, features):
    raise NotImplementedError("write your pallas kernel here")


# speedup vs baseline: 13.9292x; 13.9292x over previous
import functools

import jax
import jax.numpy as jnp
from jax import lax
from jax.experimental import pallas as pl
from jax.experimental.pallas import tpu as pltpu
from jax.experimental.pallas import tpu_sc as plsc

B, N, S, C = 4, 8192, 1024, 128
NS = 32
R2 = 0.2 * 0.2
CH = 3 + C
L = 16
NCORES, NSUB = 2, 16
QPW = (B * S) // (NCORES * NSUB)
GCHUNKS = (S * NS) // L


def _body(aug, nxt, out, idx_hbm,
          xb, yb, zb, qxb, qyb, qzb, qrow, idx_blk,
          row_vm, out_vm, cent_vm, idx_vm):
    c = lax.axis_index("c")
    t = lax.axis_index("s")

    b1 = c * 2 + t // 8
    qs0 = (t % 8) * QPW
    pltpu.sync_copy(aug.at[pl.ds((b1 * CH + 0) * N, N)], xb)
    pltpu.sync_copy(aug.at[pl.ds((b1 * CH + 1) * N, N)], yb)
    pltpu.sync_copy(aug.at[pl.ds((b1 * CH + 2) * N, N)], zb)
    pltpu.sync_copy(nxt.at[pl.ds((b1 * 3 + 0) * S + qs0, QPW)],
                    qxb.at[pl.ds(0, QPW)])
    pltpu.sync_copy(nxt.at[pl.ds((b1 * 3 + 1) * S + qs0, QPW)],
                    qyb.at[pl.ds(0, QPW)])
    pltpu.sync_copy(nxt.at[pl.ds((b1 * 3 + 2) * S + qs0, QPW)],
                    qzb.at[pl.ds(0, QPW)])

    iota = lax.iota(jnp.int32, L)
    zidx = jnp.zeros((L,), jnp.int32)

    def per_query(q, carry):
        qx = jnp.full((L,), qxb[pl.ds(q, L)][0], jnp.float32)
        qy = jnp.full((L,), qyb[pl.ds(q, L)][0], jnp.float32)
        qz = jnp.full((L,), qzb[pl.ds(q, L)][0], jnp.float32)
        qrow[pl.ds(0, L)] = jnp.zeros((L,), jnp.int32)

        def cond(nc):
            n, cnt = nc
            return (cnt < NS) & (n < N)

        def step(nc):
            n, cnt = nc
            dx = xb[pl.ds(n, L)] - qx
            dy = yb[pl.ds(n, L)] - qy
            dz = zb[pl.ds(n, L)] - qz
            d2 = dx * dx + dy * dy + dz * dz
            m = d2 < R2
            plsc.store_compressed(qrow.at[pl.ds(cnt, L)], iota + n, mask=m)
            return (n + L, cnt + plsc.all_reduce_population_count(m)[0])

        _, cnt = lax.while_loop(cond, step, (jnp.int32(0), jnp.int32(0)))
        found = jnp.minimum(cnt, NS)
        v0 = qrow[pl.ds(0, L)]
        v1 = qrow[pl.ds(L, L)]
        fv = plsc.load_gather(qrow, [zidx])
        v0 = jnp.where(iota < found, v0, fv)
        v1 = jnp.where(iota + L < found, v1, fv)
        idx_blk[pl.ds(q * NS, L)] = v0
        idx_blk[pl.ds(q * NS + L, L)] = v1
        return carry

    lax.fori_loop(0, QPW, per_query, 0)
    pltpu.sync_copy(idx_blk,
                    idx_hbm.at[pl.ds((b1 * S + qs0) * NS, QPW * NS)])
    plsc.subcore_barrier()

    for bl in range(2):
        b = c * 2 + bl
        pltpu.sync_copy(idx_hbm.at[pl.ds(b * S * NS, S * NS)], idx_vm)
        for r in range((CH + NSUB - 1) // NSUB):
            ch = r * NSUB + t

            @pl.when(ch < CH)
            def _():
                pltpu.sync_copy(aug.at[pl.ds((b * CH + ch) * N, N)], row_vm)

                def gat(i, carry):
                    j = i * L
                    iv = idx_vm[pl.ds(j, L)]
                    out_vm[pl.ds(j, L)] = plsc.load_gather(row_vm, [iv])
                    return carry

                lax.fori_loop(0, GCHUNKS, gat, 0)

                @pl.when(ch < 3)
                def _():
                    pltpu.sync_copy(nxt.at[pl.ds((b * 3 + ch) * S, S)],
                                    cent_vm.at[pl.ds(0, S)])

                    def fix(i, carry):
                        j = i * L
                        cv = jnp.full((L,), cent_vm[pl.ds(j // NS, L)][0],
                                      jnp.float32)
                        out_vm[pl.ds(j, L)] = out_vm[pl.ds(j, L)] - cv
                        return carry

                    lax.fori_loop(0, GCHUNKS, fix, 0)

                pltpu.sync_copy(out_vm,
                                out.at[pl.ds((b * CH + ch) * S * NS, S * NS)])


_grouper = functools.partial(
    pl.kernel,
    out_type=(
        jax.ShapeDtypeStruct((B * CH * S * NS,), jnp.float32),
        jax.ShapeDtypeStruct((B * S * NS,), jnp.int32),
    ),
    mesh=plsc.VectorSubcoreMesh(core_axis_name="c", subcore_axis_name="s"),
    compiler_params=pltpu.CompilerParams(needs_layout_passes=False),
    scratch_types=[
        pltpu.VMEM((N,), jnp.float32),
        pltpu.VMEM((N,), jnp.float32),
        pltpu.VMEM((N,), jnp.float32),
        pltpu.VMEM((QPW + L,), jnp.float32),
        pltpu.VMEM((QPW + L,), jnp.float32),
        pltpu.VMEM((QPW + L,), jnp.float32),
        pltpu.VMEM((NS + L * 2,), jnp.int32),
        pltpu.VMEM((QPW * NS,), jnp.int32),
        pltpu.VMEM((N,), jnp.float32),
        pltpu.VMEM((S * NS,), jnp.float32),
        pltpu.VMEM((S + L,), jnp.float32),
        pltpu.VMEM((S * NS,), jnp.int32),
    ],
)(_body)


def kernel(xyz, new_xyz, features):
    xyzT = jnp.transpose(xyz, (0, 2, 1))
    aug = jnp.concatenate([xyzT, features], axis=1).reshape(-1)
    nxt = jnp.transpose(new_xyz, (0, 2, 1)).reshape(-1)
    out, _ = _grouper(aug, nxt)
    return out.reshape(B, CH, S, NS)


# speedup vs baseline: 21.7147x; 1.5589x over previous
import functools

import jax
import jax.numpy as jnp
from jax import lax
from jax.experimental import pallas as pl
from jax.experimental.pallas import tpu as pltpu
from jax.experimental.pallas import tpu_sc as plsc

B, N, S, C = 4, 8192, 1024, 128
NS = 32
R2 = 0.2 * 0.2
CH = 3 + C
L = 16
NCORES, NSUB = 2, 16
QPW = (B * S) // (NCORES * NSUB)
GCHUNKS = (S * NS) // L
PPI = 4 * L
QROW = NS + 4 * L


def _body(xyzT, nxt, feat, out, idx_hbm,
          xb, yb, zb, qxb, qyb, qzb, qrow, idx_blk, gxyz,
          row_vm, out_vm, idx_vm):
    c = lax.axis_index("c")
    t = lax.axis_index("s")

    b1 = c * 2 + t // 8
    qs0 = (t % 8) * QPW
    pltpu.sync_copy(xyzT.at[pl.ds((b1 * 3 + 0) * N, N)], xb)
    pltpu.sync_copy(xyzT.at[pl.ds((b1 * 3 + 1) * N, N)], yb)
    pltpu.sync_copy(xyzT.at[pl.ds((b1 * 3 + 2) * N, N)], zb)
    pltpu.sync_copy(nxt.at[pl.ds((b1 * 3 + 0) * S + qs0, QPW)],
                    qxb.at[pl.ds(0, QPW)])
    pltpu.sync_copy(nxt.at[pl.ds((b1 * 3 + 1) * S + qs0, QPW)],
                    qyb.at[pl.ds(0, QPW)])
    pltpu.sync_copy(nxt.at[pl.ds((b1 * 3 + 2) * S + qs0, QPW)],
                    qzb.at[pl.ds(0, QPW)])

    iota = lax.iota(jnp.int32, L)
    zidx = jnp.zeros((L,), jnp.int32)

    def per_query(q, carry):
        qx = jnp.full((L,), qxb[pl.ds(q, L)][0], jnp.float32)
        qy = jnp.full((L,), qyb[pl.ds(q, L)][0], jnp.float32)
        qz = jnp.full((L,), qzb[pl.ds(q, L)][0], jnp.float32)
        qrow[pl.ds(0, L)] = jnp.zeros((L,), jnp.int32)

        def cond(nc):
            n, cnt = nc
            return (cnt < NS) & (n < N)

        def step(nc):
            n, cnt = nc
            o = cnt
            for k in range(PPI // L):
                nk = n + k * L
                dx = xb[pl.ds(nk, L)] - qx
                dy = yb[pl.ds(nk, L)] - qy
                dz = zb[pl.ds(nk, L)] - qz
                d2 = dx * dx + dy * dy + dz * dz
                m = d2 < R2
                plsc.store_compressed(qrow.at[pl.ds(o, L)], iota + nk, mask=m)
                o = o + plsc.all_reduce_population_count(m)[0]
            return (n + PPI, o)

        _, cnt = lax.while_loop(cond, step, (jnp.int32(0), jnp.int32(0)))
        found = jnp.minimum(cnt, NS)
        v0 = qrow[pl.ds(0, L)]
        v1 = qrow[pl.ds(L, L)]
        fv = plsc.load_gather(qrow, [zidx])
        v0 = jnp.where(iota < found, v0, fv)
        v1 = jnp.where(iota + L < found, v1, fv)
        idx_blk[pl.ds(q * NS, L)] = v0
        idx_blk[pl.ds(q * NS + L, L)] = v1
        for d, (buf, qv) in enumerate(((xb, qx), (yb, qy), (zb, qz))):
            gxyz[pl.ds(d * QPW * NS + q * NS, L)] = (
                plsc.load_gather(buf, [v0]) - qv)
            gxyz[pl.ds(d * QPW * NS + q * NS + L, L)] = (
                plsc.load_gather(buf, [v1]) - qv)
        return carry

    lax.fori_loop(0, QPW, per_query, 0)
    pltpu.sync_copy(idx_blk,
                    idx_hbm.at[pl.ds((b1 * S + qs0) * NS, QPW * NS)])
    for d in range(3):
        pltpu.sync_copy(
            gxyz.at[pl.ds(d * QPW * NS, QPW * NS)],
            out.at[pl.ds(((b1 * CH + d) * S + qs0) * NS, QPW * NS)])
    plsc.subcore_barrier()

    for bl in range(2):
        b = c * 2 + bl
        pltpu.sync_copy(idx_hbm.at[pl.ds(b * S * NS, S * NS)], idx_vm)
        for r in range(C // NSUB):
            ch = r * NSUB + t
            pltpu.sync_copy(feat.at[pl.ds((b * C + ch) * N, N)], row_vm)

            @plsc.parallel_loop(0, GCHUNKS, unroll=8)
            def gat(i):
                j = i * L
                iv = idx_vm[pl.ds(j, L)]
                out_vm[pl.ds(j, L)] = plsc.load_gather(row_vm, [iv])

            pltpu.sync_copy(
                out_vm,
                out.at[pl.ds(((b * CH + 3 + ch) * S) * NS, S * NS)])


_grouper = functools.partial(
    pl.kernel,
    out_type=(
        jax.ShapeDtypeStruct((B * CH * S * NS,), jnp.float32),
        jax.ShapeDtypeStruct((B * S * NS,), jnp.int32),
    ),
    mesh=plsc.VectorSubcoreMesh(core_axis_name="c", subcore_axis_name="s"),
    compiler_params=pltpu.CompilerParams(needs_layout_passes=False),
    scratch_types=[
        pltpu.VMEM((N,), jnp.float32),
        pltpu.VMEM((N,), jnp.float32),
        pltpu.VMEM((N,), jnp.float32),
        pltpu.VMEM((QPW + L,), jnp.float32),
        pltpu.VMEM((QPW + L,), jnp.float32),
        pltpu.VMEM((QPW + L,), jnp.float32),
        pltpu.VMEM((QROW,), jnp.int32),
        pltpu.VMEM((QPW * NS,), jnp.int32),
        pltpu.VMEM((3 * QPW * NS,), jnp.float32),
        pltpu.VMEM((N,), jnp.float32),
        pltpu.VMEM((S * NS,), jnp.float32),
        pltpu.VMEM((S * NS,), jnp.int32),
    ],
)(_body)


def kernel(xyz, new_xyz, features):
    xyzT = jnp.transpose(xyz, (0, 2, 1)).reshape(-1)
    nxt = jnp.transpose(new_xyz, (0, 2, 1)).reshape(-1)
    out, _ = _grouper(xyzT, nxt, features.reshape(-1))
    return out.reshape(B, CH, S, NS)


# speedup vs baseline: 80.1396x; 3.6906x over previous
import functools

import jax
import jax.numpy as jnp
from jax import lax
from jax.experimental import pallas as pl
from jax.experimental.pallas import tpu as pltpu
from jax.experimental.pallas import tpu_sc as plsc

B, N, S, C = 4, 8192, 1024, 128
NS = 32
R2 = 0.2 * 0.2
CH = 3 + C
L = 16
NCORES, NSUB = 2, 16
QPW = (B * S) // (NCORES * NSUB)
GCHUNKS = (S * NS) // L
HCHUNKS = GCHUNKS // 2
HALF = S * NS // 2
BCHUNKS = 16
BPI = BCHUNKS * L
QROW = NS + BPI + L


def _body(xyzT, nxt, feat, out, idx_hbm,
          xb, yb, zb, qxb, qyb, qzb, qrow, idx_blk, gxyz,
          row_a, row_b, out_vm, idx_vm, sem_a, sem_b, sem_o0, sem_o1):
    c = lax.axis_index("c")
    t = lax.axis_index("s")

    b1 = c * 2 + t // 8
    qs0 = (t % 8) * QPW
    pltpu.sync_copy(xyzT.at[pl.ds((b1 * 3 + 0) * N, N)], xb)
    pltpu.sync_copy(xyzT.at[pl.ds((b1 * 3 + 1) * N, N)], yb)
    pltpu.sync_copy(xyzT.at[pl.ds((b1 * 3 + 2) * N, N)], zb)
    pltpu.sync_copy(nxt.at[pl.ds((b1 * 3 + 0) * S + qs0, QPW)],
                    qxb.at[pl.ds(0, QPW)])
    pltpu.sync_copy(nxt.at[pl.ds((b1 * 3 + 1) * S + qs0, QPW)],
                    qyb.at[pl.ds(0, QPW)])
    pltpu.sync_copy(nxt.at[pl.ds((b1 * 3 + 2) * S + qs0, QPW)],
                    qzb.at[pl.ds(0, QPW)])
    b2 = c * 2
    cp_row = pltpu.async_copy(feat.at[pl.ds(b2 * C + t, 1), :], row_a, sem_a)

    iota = lax.iota(jnp.int32, L)
    zidx = jnp.zeros((L,), jnp.int32)

    def per_query(q, carry):
        qx = jnp.full((L,), qxb[pl.ds(q, L)][0], jnp.float32)
        qy = jnp.full((L,), qyb[pl.ds(q, L)][0], jnp.float32)
        qz = jnp.full((L,), qzb[pl.ds(q, L)][0], jnp.float32)
        qrow[pl.ds(0, L)] = jnp.zeros((L,), jnp.int32)

        def cond(nc):
            n, ov = nc
            return (ov[0] < NS - 1) & (n < N)

        def step(nc):
            n, ov = nc

            def blk(k, o):
                nk = n + k * L
                dx = xb[pl.ds(nk, L)] - qx
                dy = yb[pl.ds(nk, L)] - qy
                dz = zb[pl.ds(nk, L)] - qz
                d2 = dx * dx + dy * dy + dz * dz
                m = d2 < R2
                slot = o + plsc.cumsum(m.astype(jnp.int32))
                plsc.store_scatter(qrow, [slot], iota + nk, mask=m)
                return o + plsc.all_reduce_population_count(m)

            ov2 = plsc.parallel_loop(0, BCHUNKS, unroll=8, carry=ov)(blk)
            return (n + BPI, ov2)

        _, ovf = lax.while_loop(
            cond, step, (jnp.int32(0), jnp.full((L,), -1, jnp.int32)))
        found = jnp.minimum(ovf[0] + 1, NS)
        v0 = qrow[pl.ds(0, L)]
        v1 = qrow[pl.ds(L, L)]
        fv = plsc.load_gather(qrow, [zidx])
        v0 = jnp.where(iota < found, v0, fv)
        v1 = jnp.where(iota + L < found, v1, fv)
        qsplat = jnp.full((L,), q, jnp.int32)
        plsc.store_scatter(idx_blk, [iota, qsplat], v0)
        plsc.store_scatter(idx_blk, [L + iota, qsplat], v1)
        for d, (buf, qv) in enumerate(((xb, qx), (yb, qy), (zb, qz))):
            plsc.store_scatter(gxyz, [d * NS + iota, qsplat],
                               plsc.load_gather(buf, [v0]) - qv)
            plsc.store_scatter(gxyz, [d * NS + L + iota, qsplat],
                               plsc.load_gather(buf, [v1]) - qv)
        return carry

    lax.fori_loop(0, QPW, per_query, 0)
    pltpu.sync_copy(idx_blk,
                    idx_hbm.at[pl.ds(b1 * NS, NS), pl.ds(qs0, QPW)])
    for d in range(3):
        pltpu.sync_copy(
            gxyz.at[pl.ds(d * NS, NS), :],
            out.at[pl.ds((b1 * CH + d) * NS, NS), pl.ds(qs0, QPW)])
    plsc.subcore_barrier()

    rows = (row_a, row_b)
    row_sems = (sem_a, sem_b)
    out_cp = [None, None]
    for i in range(2 * C // NSUB):
        bl, r = divmod(i, C // NSUB)
        b = c * 2 + bl
        ch = r * NSUB + t
        if r == 0:
            pltpu.sync_copy(idx_hbm.at[pl.ds(b * NS, NS), :], idx_vm)
        if i + 1 < 2 * C // NSUB:
            nbl, nr = divmod(i + 1, C // NSUB)
            nch = nr * NSUB + t
            nrow = (c * 2 + nbl) * C + nch
            cp_next = pltpu.async_copy(
                feat.at[pl.ds(nrow, 1), :], rows[(i + 1) % 2],
                row_sems[(i + 1) % 2])
        else:
            cp_next = None
        cp_row.wait()
        cur = rows[i % 2]
        row0 = (b * CH + 3 + ch) * NS
        for h, sem_o in enumerate((sem_o0, sem_o1)):
            if out_cp[h] is not None:
                out_cp[h].wait()

            @plsc.parallel_loop(0, HCHUNKS, unroll=8)
            def gat(g):
                k = h * (NS // 2) + g // (S // L)
                s0 = (g % (S // L)) * L
                iv = idx_vm[k, pl.ds(s0, L)]
                out_vm[k, pl.ds(s0, L)] = plsc.load_gather(cur, [zidx, iv])

            out_cp[h] = pltpu.async_copy(
                out_vm.at[pl.ds(h * (NS // 2), NS // 2), :],
                out.at[pl.ds(row0 + h * (NS // 2), NS // 2), :],
                sem_o)
        cp_row = cp_next
    out_cp[0].wait()
    out_cp[1].wait()


_grouper = functools.partial(
    pl.kernel,
    out_type=(
        jax.ShapeDtypeStruct((B * CH * NS, S), jnp.float32),
        jax.ShapeDtypeStruct((B * NS, S), jnp.int32),
    ),
    mesh=plsc.VectorSubcoreMesh(core_axis_name="c", subcore_axis_name="s"),
    compiler_params=pltpu.CompilerParams(needs_layout_passes=False),
    scratch_types=[
        pltpu.VMEM((N,), jnp.float32),
        pltpu.VMEM((N,), jnp.float32),
        pltpu.VMEM((N,), jnp.float32),
        pltpu.VMEM((QPW + L,), jnp.float32),
        pltpu.VMEM((QPW + L,), jnp.float32),
        pltpu.VMEM((QPW + L,), jnp.float32),
        pltpu.VMEM((QROW,), jnp.int32),
        pltpu.VMEM((NS, QPW), jnp.int32),
        pltpu.VMEM((3 * NS, QPW), jnp.float32),
        pltpu.VMEM((1, N), jnp.float32),
        pltpu.VMEM((1, N), jnp.float32),
        pltpu.VMEM((NS, S), jnp.float32),
        pltpu.VMEM((NS, S), jnp.int32),
        pltpu.SemaphoreType.DMA,
        pltpu.SemaphoreType.DMA,
        pltpu.SemaphoreType.DMA,
        pltpu.SemaphoreType.DMA,
    ],
)(_body)


def kernel(xyz, new_xyz, features):
    xyzT = jnp.transpose(xyz, (0, 2, 1)).reshape(-1)
    nxt = jnp.transpose(new_xyz, (0, 2, 1)).reshape(-1)
    out, _ = _grouper(xyzT, nxt, features.reshape(B * C, N))
    return out.reshape(B, CH, NS, S).swapaxes(2, 3)


# speedup vs baseline: 87.0349x; 1.0860x over previous
import functools

import jax
import jax.numpy as jnp
from jax import lax
from jax.experimental import pallas as pl
from jax.experimental.pallas import tpu as pltpu
from jax.experimental.pallas import tpu_sc as plsc

B, N, S, C = 4, 8192, 1024, 128
NS = 32
R2 = 0.2 * 0.2
CH = 3 + C
L = 16
NCORES, NSUB = 2, 16
QPW = (B * S) // (NCORES * NSUB)
GCHUNKS = (S * NS) // L
HCHUNKS = GCHUNKS // 2
HALF = S * NS // 2
BCHUNKS = 16
BPI = BCHUNKS * L
QROW = NS + BPI + L


def _body(xyzT, nxt, feat, out, idx_hbm,
          xb, yb, zb, qxb, qyb, qzb, qrow, idx_blk, gxyz,
          row_a, row_b, out_vm, idx_vm, sem_a, sem_b, sem_o0, sem_o1):
    c = lax.axis_index("c")
    t = lax.axis_index("s")

    b1 = c * 2 + t // 8
    qs0 = (t % 8) * QPW
    pltpu.sync_copy(xyzT.at[pl.ds((b1 * 3 + 0) * N, N)], xb)
    pltpu.sync_copy(xyzT.at[pl.ds((b1 * 3 + 1) * N, N)], yb)
    pltpu.sync_copy(xyzT.at[pl.ds((b1 * 3 + 2) * N, N)], zb)
    pltpu.sync_copy(nxt.at[pl.ds((b1 * 3 + 0) * S + qs0, QPW)],
                    qxb.at[pl.ds(0, QPW)])
    pltpu.sync_copy(nxt.at[pl.ds((b1 * 3 + 1) * S + qs0, QPW)],
                    qyb.at[pl.ds(0, QPW)])
    pltpu.sync_copy(nxt.at[pl.ds((b1 * 3 + 2) * S + qs0, QPW)],
                    qzb.at[pl.ds(0, QPW)])
    b2 = c * 2
    cp_row = pltpu.async_copy(feat.at[pl.ds(b2 * C + t, 1), :], row_a, sem_a)

    iota = lax.iota(jnp.int32, L)
    zidx = jnp.zeros((L,), jnp.int32)

    def per_query(q, carry):
        qx = jnp.full((L,), qxb[pl.ds(q, L)][0], jnp.float32)
        qy = jnp.full((L,), qyb[pl.ds(q, L)][0], jnp.float32)
        qz = jnp.full((L,), qzb[pl.ds(q, L)][0], jnp.float32)
        qrow[pl.ds(0, L)] = jnp.zeros((L,), jnp.int32)

        def cond(nc):
            n, ov = nc
            return (ov[0] < NS - 1) & (n < N)

        def step(nc):
            n, ov = nc

            def blk(k, o):
                nk = n + k * L
                dx = xb[pl.ds(nk, L)] - qx
                dy = yb[pl.ds(nk, L)] - qy
                dz = zb[pl.ds(nk, L)] - qz
                d2 = dx * dx + dy * dy + dz * dz
                m = d2 < R2
                slot = o + plsc.cumsum(m.astype(jnp.int32))
                plsc.store_scatter(qrow, [slot], iota + nk, mask=m)
                return o + plsc.all_reduce_population_count(m)

            ov2 = plsc.parallel_loop(0, BCHUNKS, unroll=4, carry=ov)(blk)
            return (n + BPI, ov2)

        _, ovf = lax.while_loop(
            cond, step, (jnp.int32(0), jnp.full((L,), -1, jnp.int32)))
        found = jnp.minimum(ovf[0] + 1, NS)
        v0 = qrow[pl.ds(0, L)]
        v1 = qrow[pl.ds(L, L)]
        fv = plsc.load_gather(qrow, [zidx])
        v0 = jnp.where(iota < found, v0, fv)
        v1 = jnp.where(iota + L < found, v1, fv)
        qsplat = jnp.full((L,), q, jnp.int32)
        plsc.store_scatter(idx_blk, [iota, qsplat], v0)
        plsc.store_scatter(idx_blk, [L + iota, qsplat], v1)
        for d, (buf, qv) in enumerate(((xb, qx), (yb, qy), (zb, qz))):
            plsc.store_scatter(gxyz, [d * NS + iota, qsplat],
                               plsc.load_gather(buf, [v0]) - qv)
            plsc.store_scatter(gxyz, [d * NS + L + iota, qsplat],
                               plsc.load_gather(buf, [v1]) - qv)
        return carry

    lax.fori_loop(0, QPW, per_query, 0)
    pltpu.sync_copy(idx_blk,
                    idx_hbm.at[pl.ds(b1 * NS, NS), pl.ds(qs0, QPW)])
    for d in range(3):
        pltpu.sync_copy(
            gxyz.at[pl.ds(d * NS, NS), :],
            out.at[pl.ds((b1 * CH + d) * NS, NS), pl.ds(qs0, QPW)])
    plsc.subcore_barrier()

    rows = (row_a, row_b)
    row_sems = (sem_a, sem_b)
    out_cp = [None, None]
    for i in range(2 * C // NSUB):
        bl, r = divmod(i, C // NSUB)
        b = c * 2 + bl
        ch = r * NSUB + t
        if r == 0:
            pltpu.sync_copy(idx_hbm.at[pl.ds(b * NS, NS), :], idx_vm)
        if i + 1 < 2 * C // NSUB:
            nbl, nr = divmod(i + 1, C // NSUB)
            nch = nr * NSUB + t
            nrow = (c * 2 + nbl) * C + nch
            cp_next = pltpu.async_copy(
                feat.at[pl.ds(nrow, 1), :], rows[(i + 1) % 2],
                row_sems[(i + 1) % 2])
        else:
            cp_next = None
        cp_row.wait()
        cur = rows[i % 2]
        row0 = (b * CH + 3 + ch) * NS
        for h, sem_o in enumerate((sem_o0, sem_o1)):
            if out_cp[h] is not None:
                out_cp[h].wait()

            @plsc.parallel_loop(0, HCHUNKS, unroll=8)
            def gat(g):
                k = h * (NS // 2) + g // (S // L)
                s0 = (g % (S // L)) * L
                iv = idx_vm[k, pl.ds(s0, L)]
                out_vm[k, pl.ds(s0, L)] = plsc.load_gather(cur, [zidx, iv])

            out_cp[h] = pltpu.async_copy(
                out_vm.at[pl.ds(h * (NS // 2), NS // 2), :],
                out.at[pl.ds(row0 + h * (NS // 2), NS // 2), :],
                sem_o)
        cp_row = cp_next
    out_cp[0].wait()
    out_cp[1].wait()


_grouper = functools.partial(
    pl.kernel,
    out_type=(
        jax.ShapeDtypeStruct((B * CH * NS, S), jnp.float32),
        jax.ShapeDtypeStruct((B * NS, S), jnp.int32),
    ),
    mesh=plsc.VectorSubcoreMesh(core_axis_name="c", subcore_axis_name="s"),
    compiler_params=pltpu.CompilerParams(needs_layout_passes=False),
    scratch_types=[
        pltpu.VMEM((N,), jnp.float32),
        pltpu.VMEM((N,), jnp.float32),
        pltpu.VMEM((N,), jnp.float32),
        pltpu.VMEM((QPW + L,), jnp.float32),
        pltpu.VMEM((QPW + L,), jnp.float32),
        pltpu.VMEM((QPW + L,), jnp.float32),
        pltpu.VMEM((QROW,), jnp.int32),
        pltpu.VMEM((NS, QPW), jnp.int32),
        pltpu.VMEM((3 * NS, QPW), jnp.float32),
        pltpu.VMEM((1, N), jnp.float32),
        pltpu.VMEM((1, N), jnp.float32),
        pltpu.VMEM((NS, S), jnp.float32),
        pltpu.VMEM((NS, S), jnp.int32),
        pltpu.SemaphoreType.DMA,
        pltpu.SemaphoreType.DMA,
        pltpu.SemaphoreType.DMA,
        pltpu.SemaphoreType.DMA,
    ],
)(_body)


def kernel(xyz, new_xyz, features):
    xyzT = jnp.transpose(xyz, (0, 2, 1)).reshape(-1)
    nxt = jnp.transpose(new_xyz, (0, 2, 1)).reshape(-1)
    out, _ = _grouper(xyzT, nxt, features.reshape(B * C, N))
    return out.reshape(B, CH, NS, S).swapaxes(2, 3)
